# baseline (speedup 1.0000x reference)
import functools

import jax
import jax.numpy as jnp
from jax import lax
from jax.experimental import pallas as pl
from jax.experimental.pallas import tpu as pltpu
from jax.experimental.pallas import tpu_sc as plsc

L = 16
NTILES = 16
NCORES = 2
CHUNK_ROWS = 8
ROW_W = 128


_SC_PARAMS = pltpu.CompilerParams(use_tc_tiling_on_sc=False)


def _mesh():
  return plsc.VectorSubcoreMesh(core_axis_name="c", subcore_axis_name="s")


def _make_deg(NP, EP, ZB):
  rows_per_worker = EP // ROW_W // (NCORES * NTILES)
  chunks = rows_per_worker // CHUNK_ROWS
  per_tile = NP // NTILES
  nz = per_tile // ZB

  def body(dst_r, dega, degb, didx, rows, zbuf, acc, ssem):
    c = lax.axis_index("c")
    t = lax.axis_index("s")

    def zfill(i, carry):
      zbuf[i] = jnp.zeros((L,), jnp.float32)
      return carry

    lax.fori_loop(0, ZB, zfill, 0)

    def ofill(i, carry):
      def inner(j, carry2):
        rows[i, j] = jnp.ones((L,), jnp.float32)
        return carry2
      return lax.fori_loop(0, ROW_W, inner, carry)

    lax.fori_loop(0, CHUNK_ROWS, ofill, 0)

    for k in range(nz):
      pltpu.sync_copy(zbuf, acc.at[pl.ds(t * per_tile + k * ZB, ZB)])
    plsc.subcore_barrier()

    base = (c * NTILES + t) * rows_per_worker

    def chunk(i, carry):
      r0 = base + i * CHUNK_ROWS
      pltpu.sync_copy(dst_r.at[pl.ds(r0, CHUNK_ROWS)], didx)
      descs = [
          pltpu.async_copy(rows.at[j], acc.at[didx.at[j]], ssem, add=True)
          for j in range(CHUNK_ROWS)
      ]
      for d in descs:
        d.wait()
      return carry

    lax.fori_loop(0, chunks, chunk, 0)
    plsc.subcore_barrier()

    @pl.when(c == 0)
    def _():
      pltpu.sync_copy(acc.at[pl.ds(t * per_tile, per_tile)],
                      dega.at[pl.ds(t * per_tile, per_tile)])

    @pl.when(c == 1)
    def _():
      pltpu.sync_copy(acc.at[pl.ds(t * per_tile, per_tile)],
                      degb.at[pl.ds(t * per_tile, per_tile)])

  out = [jax.ShapeDtypeStruct((NP, L), jnp.float32)] * 2
  scratch = [
      pltpu.VMEM((CHUNK_ROWS, ROW_W), jnp.int32),
      pltpu.VMEM((CHUNK_ROWS, ROW_W, L), jnp.float32),
      pltpu.VMEM((ZB, L), jnp.float32),
      pltpu.VMEM_SHARED((NP, L), jnp.float32),
      pltpu.SemaphoreType.DMA,
  ]
  return pl.kernel(body, out_type=out, mesh=_mesh(), scratch_types=scratch,
                   compiler_params=_SC_PARAMS)


def _make_hop(S, NP, EP, ZB, want_tables):
  CR = 4
  SB = 128
  rows_per_tile = EP // ROW_W // NTILES
  npairs = rows_per_tile // CR // 2
  per_tile = NP // NTILES
  nz = per_tile // ZB
  nb = per_tile // SB

  def body(src_r, dst_r, inv_h, *rest):
    rs = rest[:S]
    q = rest[S]
    tabs = rest[S + 1:S + 1 + S] if want_tables else []
    (sidx0, didx0, rows0, sidx1, didx1, rows1, zbuf, stage, istage, acc,
     gsem0, gsem1, ssem0, ssem1) = rest[S + 1 + len(tabs):]
    c = lax.axis_index("c")
    t = lax.axis_index("s")

    def zfill(i, carry):
      zbuf[i] = jnp.zeros((L,), jnp.float32)
      return carry

    lax.fori_loop(0, ZB, zfill, 0)

    for s in range(S):
      @pl.when(c == (s % NCORES))
      def _(s=s):
        for k in range(nz):
          pltpu.sync_copy(zbuf, acc.at[pl.ds(t * per_tile + k * ZB, ZB)])
        plsc.subcore_barrier()
        base = t * rows_per_tile

        def g_issue(sidx, didx, rows, gsem, k):
          e0 = (base + k * CR) * ROW_W
          pltpu.sync_copy(src_r.at[pl.ds(e0, CR * ROW_W)], sidx)
          pltpu.sync_copy(dst_r.at[pl.ds(e0, CR * ROW_W)], didx)
          pltpu.async_copy(rs[s].at[sidx], rows, gsem)

        def g_wait(sidx, rows, gsem):
          pltpu.make_async_copy(rs[s].at[sidx], rows, gsem).wait()

        def s_issue(didx, rows, ssem):
          pltpu.async_copy(rows, acc.at[didx], ssem, add=True)

        def s_wait(didx, rows, ssem):
          pltpu.make_async_copy(rows, acc.at[didx], ssem).wait()

        g_issue(sidx0, didx0, rows0, gsem0, 0)
        g_issue(sidx1, didx1, rows1, gsem1, 1)

        def pair(i, carry):
          g_wait(sidx0, rows0, gsem0)
          s_issue(didx0, rows0, ssem0)
          g_wait(sidx1, rows1, gsem1)
          s_issue(didx1, rows1, ssem1)
          s_wait(didx0, rows0, ssem0)
          g_issue(sidx0, didx0, rows0, gsem0, 2 * i + 2)
          s_wait(didx1, rows1, ssem1)
          g_issue(sidx1, didx1, rows1, gsem1, 2 * i + 3)
          return carry

        lax.fori_loop(0, npairs - 1, pair, 0)
        g_wait(sidx0, rows0, gsem0)
        s_issue(didx0, rows0, ssem0)
        g_wait(sidx1, rows1, gsem1)
        s_issue(didx1, rows1, ssem1)
        s_wait(didx0, rows0, ssem0)
        s_wait(didx1, rows1, ssem1)
        plsc.subcore_barrier()

        def wblock(b, carry):
          lo = t * per_tile + b * SB
          pltpu.sync_copy(acc.at[pl.ds(lo, SB)], stage)
          pltpu.sync_copy(inv_h.at[pl.ds(lo, SB)], istage)

          def mul(r, carry2):
            stage[r] = stage[r] * istage[r]
            return carry2

          lax.fori_loop(0, SB, mul, 0)
          pltpu.sync_copy(stage, q.at[pl.ds(lo, SB), pl.ds(s * L, L)])
          if want_tables:
            pltpu.sync_copy(stage, tabs[s].at[pl.ds(lo, SB)])
          return carry

        lax.fori_loop(0, nb, wblock, 0)
        plsc.subcore_barrier()

  out = [jax.ShapeDtypeStruct((NP, S * L), jnp.float32)]
  if want_tables:
    out += [jax.ShapeDtypeStruct((NP, L), jnp.float32)] * S
  scratch = [
      pltpu.VMEM((CR * ROW_W,), jnp.int32),
      pltpu.VMEM((CR * ROW_W,), jnp.int32),
      pltpu.VMEM((CR * ROW_W, L), jnp.float32),
      pltpu.VMEM((CR * ROW_W,), jnp.int32),
      pltpu.VMEM((CR * ROW_W,), jnp.int32),
      pltpu.VMEM((CR * ROW_W, L), jnp.float32),
      pltpu.VMEM((ZB, L), jnp.float32),
      pltpu.VMEM((SB, L), jnp.float32),
      pltpu.VMEM((SB, L), jnp.float32),
      pltpu.VMEM_SHARED((NP, L), jnp.float32),
      pltpu.SemaphoreType.DMA,
      pltpu.SemaphoreType.DMA,
      pltpu.SemaphoreType.DMA,
      pltpu.SemaphoreType.DMA,
  ]
  return pl.kernel(body, out_type=out, mesh=_mesh(), scratch_types=scratch,
                   compiler_params=_SC_PARAMS)


BN = 1024


def _prep_body(S, dega_ref, degb_ref, xp_ref, *outs):
  d = dega_ref[...] + degb_ref[...]
  pos = d > 0
  dis = jnp.where(pos, lax.rsqrt(jnp.maximum(d, 1e-12)), 0.0)
  outs[0][...] = jnp.where(pos, jnp.sqrt(d), 0.0)
  outs[1][...] = dis * dis
  r0 = xp_ref[...] * dis[:, :1]
  for s in range(S):
    outs[2 + s][...] = r0[:, s * L:(s + 1) * L]


def _layer_body(nq, S_next, *refs):
  h_ref = refs[0]
  sdeg = refs[1][...]
  inv = refs[2][...]
  rrefs = refs[3:3 + nq]
  w_ref = refs[3 + nq]
  b_ref = refs[4 + nq]
  hout_ref = refs[5 + nq]
  acc = jnp.dot(h_ref[...], w_ref[0], preferred_element_type=jnp.float32)
  s1 = sdeg[:, :1]
  for j in range(nq):
    acc = acc + jnp.dot(rrefs[j][...] * s1, w_ref[j + 1],
                        preferred_element_type=jnp.float32)
  a = acc + b_ref[0]
  hout = jnp.where(a > 0, a, jnp.exp(jnp.minimum(a, 0.0)) - 1.0)
  hout_ref[...] = hout
  if S_next:
    r_next = hout * (sdeg * inv)[:, :1]
    for s in range(S_next):
      refs[6 + nq + s][...] = r_next[:, s * L:(s + 1) * L]


def _fc_body(h_ref, w_ref, b_ref, out_ref):
  a = jnp.dot(h_ref[...], w_ref[...], preferred_element_type=jnp.float32) \
      + b_ref[0]
  out_ref[...] = jnp.maximum(a, 0.0)


def _row_spec(shape):
  nd = len(shape)
  blk = (BN,) + shape[1:]
  return pl.BlockSpec(blk, lambda i: (i,) + (0,) * (nd - 1))


def _full_spec(shape):
  nd = len(shape)
  return pl.BlockSpec(shape, lambda i: (0,) * nd)


def _tc_call(body, ins, outs):
  grid = (ins[0].shape[0] // BN,)
  in_specs = []
  for a in ins:
    if a.shape[0] % BN == 0 and a.ndim >= 2 and a.shape[0] > BN:
      in_specs.append(_row_spec(a.shape))
    else:
      in_specs.append(_full_spec(a.shape))
  out_specs = [_row_spec(o.shape) for o in outs]
  res = pl.pallas_call(
      body,
      grid=grid,
      in_specs=in_specs,
      out_specs=out_specs,
      out_shape=outs,
  )(*ins)
  return list(res)


def kernel(x, edge_index, W1, b1, W2, b2, Wfc, bfc):
  N, D_IN = x.shape
  E = edge_index.shape[1]
  K1 = W1.shape[0] - 1
  D_H1 = W1.shape[2]
  D_H2 = W2.shape[2]

  per_tile = -(-(N + L) // (NTILES * 800)) * 800
  NP = per_tile * NTILES
  ZB = per_tile // 20
  EP = -(-E // (2 * NTILES * CHUNK_ROWS * ROW_W)) * (2 * NTILES * CHUNK_ROWS
                                                     * ROW_W)
  S1 = -(-D_IN // L)
  S2 = -(-D_H1 // L)
  D1P = S1 * L
  D2P = S2 * L

  src = edge_index[0]
  dst = edge_index[1]
  npad = EP - E
  padi = (jnp.arange(npad, dtype=jnp.int32) % L) + N
  src_1d = jnp.concatenate([src, padi])
  dst_1d = jnp.concatenate([dst, padi])
  dst_r = dst_1d.reshape(EP // ROW_W, ROW_W)

  xp = jnp.pad(x, ((0, NP - N), (0, D1P - D_IN)))
  W1p = jnp.pad(W1, ((0, 0), (0, D1P - D_IN), (0, D2P - D_H1)))
  b1p = jnp.pad(b1, (0, D2P - D_H1)).reshape(1, D2P)
  W2p = jnp.pad(W2, ((0, 0), (0, D2P - D_H1), (0, 0)))
  b2p = b2.reshape(1, D_H2)
  Wfcp = jnp.pad(Wfc, ((0, 0), (0, 48 - D_IN)))
  bfcp = jnp.pad(bfc, (0, 48 - D_IN)).reshape(1, 48)

  dega, degb = _make_deg(NP, EP, ZB)(dst_r)

  sl16 = jax.ShapeDtypeStruct((NP, L), jnp.float32)
  prep = _tc_call(functools.partial(_prep_body, S1), [dega, degb, xp],
                  [sl16, sl16] + [sl16] * S1)
  sdeg, inv = prep[0], prep[1]
  tabs0 = prep[2:]

  hop1 = _make_hop(S1, NP, EP, ZB, True)
  hop1l = _make_hop(S1, NP, EP, ZB, False)
  hop2 = _make_hop(S2, NP, EP, ZB, True)
  hop2l = _make_hop(S2, NP, EP, ZB, False)

  def propagate(hopt, hopl, tabs):
    rws = []
    cur = list(tabs)
    for j in range(K1):
      res = (hopt if j + 1 < K1 else hopl)(src_1d, dst_1d, inv, *cur)
      rws.append(res[0])
      cur = res[1:]
    return rws

  r1w = propagate(hop1, hop1l, tabs0)
  louts = _tc_call(
      functools.partial(_layer_body, K1, S2),
      [xp, sdeg, inv] + r1w + [W1p, b1p],
      [jax.ShapeDtypeStruct((NP, D2P), jnp.float32)] + [sl16] * S2)
  h1, tabs1 = louts[0], louts[1:]

  r2w = propagate(hop2, hop2l, tabs1)
  (h2,) = _tc_call(
      functools.partial(_layer_body, K1, 0),
      [h1, sdeg, inv] + r2w + [W2p, b2p],
      [jax.ShapeDtypeStruct((NP, D_H2), jnp.float32)])

  (outp,) = _tc_call(_fc_body, [h2, Wfcp, bfcp],
                     [jax.ShapeDtypeStruct((NP, 48), jnp.float32)])
  return outp[:N, :D_IN]


# speedup vs baseline: 1.0112x; 1.0112x over previous
import functools

import jax
import jax.numpy as jnp
from jax import lax
from jax.experimental import pallas as pl
from jax.experimental.pallas import tpu as pltpu
from jax.experimental.pallas import tpu_sc as plsc

L = 16
NTILES = 16
NCORES = 2
CHUNK_ROWS = 8
ROW_W = 128


_SC_PARAMS = pltpu.CompilerParams(use_tc_tiling_on_sc=False)


def _mesh():
  return plsc.VectorSubcoreMesh(core_axis_name="c", subcore_axis_name="s")


def _make_deg(NP, EP, ZB):
  rows_per_worker = EP // ROW_W // (NCORES * NTILES)
  chunks = rows_per_worker // CHUNK_ROWS
  per_tile = NP // NTILES
  nz = per_tile // ZB

  def body(dst_r, dega, degb, didx, rows, zbuf, acc, ssem):
    c = lax.axis_index("c")
    t = lax.axis_index("s")

    def zfill(i, carry):
      zbuf[i] = jnp.zeros((L,), jnp.float32)
      return carry

    lax.fori_loop(0, ZB, zfill, 0)

    def ofill(i, carry):
      def inner(j, carry2):
        rows[i, j] = jnp.ones((L,), jnp.float32)
        return carry2
      return lax.fori_loop(0, ROW_W, inner, carry)

    lax.fori_loop(0, CHUNK_ROWS, ofill, 0)

    for k in range(nz):
      pltpu.sync_copy(zbuf, acc.at[pl.ds(t * per_tile + k * ZB, ZB)])
    plsc.subcore_barrier()

    base = (c * NTILES + t) * rows_per_worker

    def chunk(i, carry):
      r0 = base + i * CHUNK_ROWS
      pltpu.sync_copy(dst_r.at[pl.ds(r0, CHUNK_ROWS)], didx)
      descs = [
          pltpu.async_copy(rows.at[j], acc.at[didx.at[j]], ssem, add=True)
          for j in range(CHUNK_ROWS)
      ]
      for d in descs:
        d.wait()
      return carry

    lax.fori_loop(0, chunks, chunk, 0)
    plsc.subcore_barrier()

    @pl.when(c == 0)
    def _():
      pltpu.sync_copy(acc.at[pl.ds(t * per_tile, per_tile)],
                      dega.at[pl.ds(t * per_tile, per_tile)])

    @pl.when(c == 1)
    def _():
      pltpu.sync_copy(acc.at[pl.ds(t * per_tile, per_tile)],
                      degb.at[pl.ds(t * per_tile, per_tile)])

  out = [jax.ShapeDtypeStruct((NP, L), jnp.float32)] * 2
  scratch = [
      pltpu.VMEM((CHUNK_ROWS, ROW_W), jnp.int32),
      pltpu.VMEM((CHUNK_ROWS, ROW_W, L), jnp.float32),
      pltpu.VMEM((ZB, L), jnp.float32),
      pltpu.VMEM_SHARED((NP, L), jnp.float32),
      pltpu.SemaphoreType.DMA,
  ]
  return pl.kernel(body, out_type=out, mesh=_mesh(), scratch_types=scratch,
                   compiler_params=_SC_PARAMS)


def _make_hop(S, NP, EP, ZB, want_tables):
  CR = 4
  SB = 128
  rows_per_tile = EP // ROW_W // NTILES
  npairs = rows_per_tile // CR // 2
  per_tile = NP // NTILES
  nz = per_tile // ZB
  nb = per_tile // SB

  def body(src_r, dst_r, inv_h, *rest):
    rs = rest[:S]
    q = rest[S]
    tabs = rest[S + 1:S + 1 + S] if want_tables else []
    (sidx0, didx0, rows0, sidx1, didx1, rows1, zbuf, stage, istage, acc,
     gsem0, gsem1, ssem0, ssem1) = rest[S + 1 + len(tabs):]
    c = lax.axis_index("c")
    t = lax.axis_index("s")

    def zfill(i, carry):
      zbuf[i] = jnp.zeros((L,), jnp.float32)
      return carry

    lax.fori_loop(0, ZB, zfill, 0)

    for s in range(S):
      @pl.when(c == (s % NCORES))
      def _(s=s):
        for k in range(nz):
          pltpu.sync_copy(zbuf, acc.at[pl.ds(t * per_tile + k * ZB, ZB)])
        plsc.subcore_barrier()
        base = t * rows_per_tile

        def g_issue(sidx, didx, rows, gsem, k):
          e0 = (base + k * CR) * ROW_W
          pltpu.sync_copy(src_r.at[pl.ds(e0, CR * ROW_W)], sidx)
          pltpu.sync_copy(dst_r.at[pl.ds(e0, CR * ROW_W)], didx)
          pltpu.async_copy(rs[s].at[sidx], rows, gsem)

        def g_wait(sidx, rows, gsem):
          pltpu.make_async_copy(rs[s].at[sidx], rows, gsem).wait()

        def s_issue(didx, rows, ssem):
          pltpu.async_copy(rows, acc.at[didx], ssem, add=True)

        def s_wait(didx, rows, ssem):
          pltpu.make_async_copy(rows, acc.at[didx], ssem).wait()

        g_issue(sidx0, didx0, rows0, gsem0, 0)
        g_issue(sidx1, didx1, rows1, gsem1, 1)

        def pair(i, carry):
          g_wait(sidx0, rows0, gsem0)
          s_issue(didx0, rows0, ssem0)
          g_wait(sidx1, rows1, gsem1)
          s_issue(didx1, rows1, ssem1)
          s_wait(didx0, rows0, ssem0)
          g_issue(sidx0, didx0, rows0, gsem0, 2 * i + 2)
          s_wait(didx1, rows1, ssem1)
          g_issue(sidx1, didx1, rows1, gsem1, 2 * i + 3)
          return carry

        lax.fori_loop(0, npairs - 1, pair, 0)
        g_wait(sidx0, rows0, gsem0)
        s_issue(didx0, rows0, ssem0)
        g_wait(sidx1, rows1, gsem1)
        s_issue(didx1, rows1, ssem1)
        s_wait(didx0, rows0, ssem0)
        s_wait(didx1, rows1, ssem1)
        plsc.subcore_barrier()

        def wblock(b, carry):
          lo = t * per_tile + b * SB
          pltpu.sync_copy(acc.at[pl.ds(lo, SB)], stage)
          pltpu.sync_copy(inv_h.at[pl.ds(lo, SB)], istage)

          def mul(r, carry2):
            stage[r] = stage[r] * istage[r]
            return carry2

          lax.fori_loop(0, SB, mul, 0)
          pltpu.sync_copy(stage, q.at[pl.ds(lo, SB), pl.ds(s * L, L)])
          if want_tables:
            pltpu.sync_copy(stage, tabs[s].at[pl.ds(lo, SB)])
          return carry

        lax.fori_loop(0, nb, wblock, 0)
        plsc.subcore_barrier()

  out = [jax.ShapeDtypeStruct((NP, S * L), jnp.float32)]
  if want_tables:
    out += [jax.ShapeDtypeStruct((NP, L), jnp.float32)] * S
  scratch = [
      pltpu.VMEM((CR * ROW_W,), jnp.int32),
      pltpu.VMEM((CR * ROW_W,), jnp.int32),
      pltpu.VMEM((CR * ROW_W, L), jnp.float32),
      pltpu.VMEM((CR * ROW_W,), jnp.int32),
      pltpu.VMEM((CR * ROW_W,), jnp.int32),
      pltpu.VMEM((CR * ROW_W, L), jnp.float32),
      pltpu.VMEM((ZB, L), jnp.float32),
      pltpu.VMEM((SB, L), jnp.float32),
      pltpu.VMEM((SB, L), jnp.float32),
      pltpu.VMEM_SHARED((NP, L), jnp.float32),
      pltpu.SemaphoreType.DMA,
      pltpu.SemaphoreType.DMA,
      pltpu.SemaphoreType.DMA,
      pltpu.SemaphoreType.DMA,
  ]
  return pl.kernel(body, out_type=out, mesh=_mesh(), scratch_types=scratch,
                   compiler_params=_SC_PARAMS)


BN = 1024


def _prep_body(S, dega_ref, degb_ref, xp_ref, *outs):
  d = dega_ref[...] + degb_ref[...]
  pos = d > 0
  dis = jnp.where(pos, lax.rsqrt(jnp.maximum(d, 1e-12)), 0.0)
  outs[0][...] = jnp.where(pos, jnp.sqrt(d), 0.0)
  outs[1][...] = dis * dis
  r0 = xp_ref[...] * dis[:, :1]
  for s in range(S):
    outs[2 + s][...] = r0[:, s * L:(s + 1) * L]


def _layer_body(nq, S_next, *refs):
  h_ref = refs[0]
  sdeg = refs[1][...]
  inv = refs[2][...]
  rrefs = refs[3:3 + nq]
  w_ref = refs[3 + nq]
  b_ref = refs[4 + nq]
  hout_ref = refs[5 + nq] if S_next else refs[7 + nq]
  acc = jnp.dot(h_ref[...], w_ref[0], preferred_element_type=jnp.float32)
  s1 = sdeg[:, :1]
  for j in range(nq):
    acc = acc + jnp.dot(rrefs[j][...] * s1, w_ref[j + 1],
                        preferred_element_type=jnp.float32)
  a = acc + b_ref[0]
  hout = jnp.where(a > 0, a, jnp.exp(jnp.minimum(a, 0.0)) - 1.0)
  if S_next:
    hout_ref[...] = hout
    r_next = hout * (sdeg * inv)[:, :1]
    for s in range(S_next):
      refs[6 + nq + s][...] = r_next[:, s * L:(s + 1) * L]
  else:
    wfc_ref = refs[5 + nq]
    bfc_ref = refs[6 + nq]
    o = jnp.dot(hout, wfc_ref[...], preferred_element_type=jnp.float32) \
        + bfc_ref[0]
    hout_ref[...] = jnp.maximum(o, 0.0)


def _fc_body(h_ref, w_ref, b_ref, out_ref):
  a = jnp.dot(h_ref[...], w_ref[...], preferred_element_type=jnp.float32) \
      + b_ref[0]
  out_ref[...] = jnp.maximum(a, 0.0)


def _row_spec(shape):
  nd = len(shape)
  blk = (BN,) + shape[1:]
  return pl.BlockSpec(blk, lambda i: (i,) + (0,) * (nd - 1))


def _full_spec(shape):
  nd = len(shape)
  return pl.BlockSpec(shape, lambda i: (0,) * nd)


def _tc_call(body, ins, outs):
  grid = (ins[0].shape[0] // BN,)
  in_specs = []
  for a in ins:
    if a.shape[0] % BN == 0 and a.ndim >= 2 and a.shape[0] > BN:
      in_specs.append(_row_spec(a.shape))
    else:
      in_specs.append(_full_spec(a.shape))
  out_specs = [_row_spec(o.shape) for o in outs]
  res = pl.pallas_call(
      body,
      grid=grid,
      in_specs=in_specs,
      out_specs=out_specs,
      out_shape=outs,
  )(*ins)
  return list(res)


def kernel(x, edge_index, W1, b1, W2, b2, Wfc, bfc):
  N, D_IN = x.shape
  E = edge_index.shape[1]
  K1 = W1.shape[0] - 1
  D_H1 = W1.shape[2]
  D_H2 = W2.shape[2]

  per_tile = -(-(N + L) // (NTILES * 800)) * 800
  NP = per_tile * NTILES
  ZB = per_tile // 20
  EP = -(-E // (2 * NTILES * CHUNK_ROWS * ROW_W)) * (2 * NTILES * CHUNK_ROWS
                                                     * ROW_W)
  S1 = -(-D_IN // L)
  S2 = -(-D_H1 // L)
  D1P = S1 * L
  D2P = S2 * L

  src = edge_index[0]
  dst = edge_index[1]
  npad = EP - E
  padi = (jnp.arange(npad, dtype=jnp.int32) % L) + N
  src_1d = jnp.concatenate([src, padi])
  dst_1d = jnp.concatenate([dst, padi])
  dst_r = dst_1d.reshape(EP // ROW_W, ROW_W)

  xp = jnp.pad(x, ((0, NP - N), (0, D1P - D_IN)))
  W1p = jnp.pad(W1, ((0, 0), (0, D1P - D_IN), (0, D2P - D_H1)))
  b1p = jnp.pad(b1, (0, D2P - D_H1)).reshape(1, D2P)
  W2p = jnp.pad(W2, ((0, 0), (0, D2P - D_H1), (0, 0)))
  b2p = b2.reshape(1, D_H2)
  Wfcp = jnp.pad(Wfc, ((0, 0), (0, 48 - D_IN)))
  bfcp = jnp.pad(bfc, (0, 48 - D_IN)).reshape(1, 48)

  dega, degb = _make_deg(NP, EP, ZB)(dst_r)

  sl16 = jax.ShapeDtypeStruct((NP, L), jnp.float32)
  prep = _tc_call(functools.partial(_prep_body, S1), [dega, degb, xp],
                  [sl16, sl16] + [sl16] * S1)
  sdeg, inv = prep[0], prep[1]
  tabs0 = prep[2:]

  hop1 = _make_hop(S1, NP, EP, ZB, True)
  hop1l = _make_hop(S1, NP, EP, ZB, False)
  hop2 = _make_hop(S2, NP, EP, ZB, True)
  hop2l = _make_hop(S2, NP, EP, ZB, False)

  def propagate(hopt, hopl, tabs):
    rws = []
    cur = list(tabs)
    for j in range(K1):
      res = (hopt if j + 1 < K1 else hopl)(src_1d, dst_1d, inv, *cur)
      rws.append(res[0])
      cur = res[1:]
    return rws

  r1w = propagate(hop1, hop1l, tabs0)
  louts = _tc_call(
      functools.partial(_layer_body, K1, S2),
      [xp, sdeg, inv] + r1w + [W1p, b1p],
      [jax.ShapeDtypeStruct((NP, D2P), jnp.float32)] + [sl16] * S2)
  h1, tabs1 = louts[0], louts[1:]

  r2w = propagate(hop2, hop2l, tabs1)
  (outp,) = _tc_call(
      functools.partial(_layer_body, K1, 0),
      [h1, sdeg, inv] + r2w + [W2p, b2p, Wfcp, bfcp],
      [jax.ShapeDtypeStruct((NP, 48), jnp.float32)])
  return outp[:N, :D_IN]


# speedup vs baseline: 1.3582x; 1.3432x over previous
import functools

import jax
import jax.numpy as jnp
from jax import lax
from jax.experimental import pallas as pl
from jax.experimental.pallas import tpu as pltpu
from jax.experimental.pallas import tpu_sc as plsc

L = 16
NTILES = 16
NCORES = 2
CHUNK_ROWS = 8
ROW_W = 128


_SC_PARAMS = pltpu.CompilerParams(use_tc_tiling_on_sc=False)


def _mesh():
  return plsc.VectorSubcoreMesh(core_axis_name="c", subcore_axis_name="s")


def _make_deg(NP, EP, ZB):
  rows_per_worker = EP // ROW_W // (NCORES * NTILES)
  chunks = rows_per_worker // CHUNK_ROWS
  per_tile = NP // NTILES
  nz = per_tile // ZB

  def body(dst_r, dega, degb, didx, rows, zbuf, acc, ssem):
    c = lax.axis_index("c")
    t = lax.axis_index("s")

    def zfill(i, carry):
      zbuf[i] = jnp.zeros((L,), jnp.float32)
      return carry

    lax.fori_loop(0, ZB, zfill, 0)

    def ofill(i, carry):
      def inner(j, carry2):
        rows[i, j] = jnp.ones((L,), jnp.float32)
        return carry2
      return lax.fori_loop(0, ROW_W, inner, carry)

    lax.fori_loop(0, CHUNK_ROWS, ofill, 0)

    for k in range(nz):
      pltpu.sync_copy(zbuf, acc.at[pl.ds(t * per_tile + k * ZB, ZB)])
    plsc.subcore_barrier()

    base = (c * NTILES + t) * rows_per_worker

    def chunk(i, carry):
      r0 = base + i * CHUNK_ROWS
      pltpu.sync_copy(dst_r.at[pl.ds(r0, CHUNK_ROWS)], didx)
      descs = [
          pltpu.async_copy(rows.at[j], acc.at[didx.at[j]], ssem, add=True)
          for j in range(CHUNK_ROWS)
      ]
      for d in descs:
        d.wait()
      return carry

    lax.fori_loop(0, chunks, chunk, 0)
    plsc.subcore_barrier()

    @pl.when(c == 0)
    def _():
      pltpu.sync_copy(acc.at[pl.ds(t * per_tile, per_tile)],
                      dega.at[pl.ds(t * per_tile, per_tile)])

    @pl.when(c == 1)
    def _():
      pltpu.sync_copy(acc.at[pl.ds(t * per_tile, per_tile)],
                      degb.at[pl.ds(t * per_tile, per_tile)])

  out = [jax.ShapeDtypeStruct((NP, L), jnp.float32)] * 2
  scratch = [
      pltpu.VMEM((CHUNK_ROWS, ROW_W), jnp.int32),
      pltpu.VMEM((CHUNK_ROWS, ROW_W, L), jnp.float32),
      pltpu.VMEM((ZB, L), jnp.float32),
      pltpu.VMEM_SHARED((NP, L), jnp.float32),
      pltpu.SemaphoreType.DMA,
  ]
  return pl.kernel(body, out_type=out, mesh=_mesh(), scratch_types=scratch,
                   compiler_params=_SC_PARAMS)


def _make_hop(S, NP, EP, ZB, want_tables):
  CR = 4
  SB = 128
  rows_per_tile = EP // ROW_W // NTILES
  npairs = rows_per_tile // CR // 2
  per_tile = NP // NTILES
  nz = per_tile // ZB
  nb = per_tile // SB

  def body(src_r, dst_r, inv_h, *rest):
    rs = rest[:S]
    q = rest[S]
    tabs = rest[S + 1:S + 1 + S] if want_tables else []
    (sidx0, didx0, rows0, sidx1, didx1, rows1, dsc0, dsc1, zbuf, stage,
     istage, acc, gsem0, gsem1, ssem0, ssem1, isem0,
     isem1) = rest[S + 1 + len(tabs):]
    c = lax.axis_index("c")
    t = lax.axis_index("s")

    def zfill(i, carry):
      zbuf[i] = jnp.zeros((L,), jnp.float32)
      return carry

    lax.fori_loop(0, ZB, zfill, 0)

    for s in range(S):
      @pl.when(c == (s % NCORES))
      def _(s=s):
        for k in range(nz):
          pltpu.sync_copy(zbuf, acc.at[pl.ds(t * per_tile + k * ZB, ZB)])
        plsc.subcore_barrier()
        base = t * rows_per_tile

        def i_issue(sidx, didx, isem, k):
          e0 = (base + k * CR) * ROW_W
          pltpu.async_copy(src_r.at[pl.ds(e0, CR * ROW_W)], sidx, isem)
          pltpu.async_copy(dst_r.at[pl.ds(e0, CR * ROW_W)], didx, isem)

        def i_wait(sidx, didx, isem):
          pltpu.make_async_copy(src_r.at[pl.ds(0, CR * ROW_W)], sidx,
                                isem).wait()
          pltpu.make_async_copy(dst_r.at[pl.ds(0, CR * ROW_W)], didx,
                                isem).wait()

        def g_fire(sidx, rows, gsem):
          pltpu.async_copy(rs[s].at[sidx], rows, gsem)

        def g_wait(sidx, rows, gsem):
          pltpu.make_async_copy(rs[s].at[sidx], rows, gsem).wait()

        def s_issue(didx, dsc, rows, ssem):
          def cbody(r, carry):
            dsc[pl.ds(r * L, L)] = didx[pl.ds(r * L, L)]
            return carry

          lax.fori_loop(0, CR * ROW_W // L, cbody, 0)
          pltpu.async_copy(rows, acc.at[dsc], ssem, add=True)

        def s_wait(dsc, rows, ssem):
          pltpu.make_async_copy(rows, acc.at[dsc], ssem).wait()

        i_issue(sidx0, didx0, isem0, 0)
        i_issue(sidx1, didx1, isem1, 1)
        i_wait(sidx0, didx0, isem0)
        g_fire(sidx0, rows0, gsem0)
        i_wait(sidx1, didx1, isem1)
        g_fire(sidx1, rows1, gsem1)

        def pair(i, carry):
          g_wait(sidx0, rows0, gsem0)
          s_issue(didx0, dsc0, rows0, ssem0)
          i_issue(sidx0, didx0, isem0, 2 * i + 2)
          g_wait(sidx1, rows1, gsem1)
          s_issue(didx1, dsc1, rows1, ssem1)
          i_issue(sidx1, didx1, isem1, 2 * i + 3)
          s_wait(dsc0, rows0, ssem0)
          i_wait(sidx0, didx0, isem0)
          g_fire(sidx0, rows0, gsem0)
          s_wait(dsc1, rows1, ssem1)
          i_wait(sidx1, didx1, isem1)
          g_fire(sidx1, rows1, gsem1)
          return carry

        lax.fori_loop(0, npairs - 1, pair, 0)
        g_wait(sidx0, rows0, gsem0)
        s_issue(didx0, dsc0, rows0, ssem0)
        g_wait(sidx1, rows1, gsem1)
        s_issue(didx1, dsc1, rows1, ssem1)
        s_wait(dsc0, rows0, ssem0)
        s_wait(dsc1, rows1, ssem1)
        plsc.subcore_barrier()

        def wblock(b, carry):
          lo = t * per_tile + b * SB
          pltpu.sync_copy(acc.at[pl.ds(lo, SB)], stage)
          pltpu.sync_copy(inv_h.at[pl.ds(lo, SB)], istage)

          def mul(r, carry2):
            stage[r] = stage[r] * istage[r]
            return carry2

          lax.fori_loop(0, SB, mul, 0)
          pltpu.sync_copy(stage, q.at[pl.ds(lo, SB), pl.ds(s * L, L)])
          if want_tables:
            pltpu.sync_copy(stage, tabs[s].at[pl.ds(lo, SB)])
          return carry

        lax.fori_loop(0, nb, wblock, 0)
        plsc.subcore_barrier()

  out = [jax.ShapeDtypeStruct((NP, S * L), jnp.float32)]
  if want_tables:
    out += [jax.ShapeDtypeStruct((NP, L), jnp.float32)] * S
  scratch = [
      pltpu.VMEM((CR * ROW_W,), jnp.int32),
      pltpu.VMEM((CR * ROW_W,), jnp.int32),
      pltpu.VMEM((CR * ROW_W, L), jnp.float32),
      pltpu.VMEM((CR * ROW_W,), jnp.int32),
      pltpu.VMEM((CR * ROW_W,), jnp.int32),
      pltpu.VMEM((CR * ROW_W, L), jnp.float32),
      pltpu.VMEM((CR * ROW_W,), jnp.int32),
      pltpu.VMEM((CR * ROW_W,), jnp.int32),
      pltpu.VMEM((ZB, L), jnp.float32),
      pltpu.VMEM((SB, L), jnp.float32),
      pltpu.VMEM((SB, L), jnp.float32),
      pltpu.VMEM_SHARED((NP, L), jnp.float32),
      pltpu.SemaphoreType.DMA,
      pltpu.SemaphoreType.DMA,
      pltpu.SemaphoreType.DMA,
      pltpu.SemaphoreType.DMA,
      pltpu.SemaphoreType.DMA,
      pltpu.SemaphoreType.DMA,
  ]
  return pl.kernel(body, out_type=out, mesh=_mesh(), scratch_types=scratch,
                   compiler_params=_SC_PARAMS)


BN = 1024


def _prep_body(S, dega_ref, degb_ref, xp_ref, *outs):
  d = dega_ref[...] + degb_ref[...]
  pos = d > 0
  dis = jnp.where(pos, lax.rsqrt(jnp.maximum(d, 1e-12)), 0.0)
  outs[0][...] = jnp.where(pos, jnp.sqrt(d), 0.0)
  outs[1][...] = dis * dis
  r0 = xp_ref[...] * dis[:, :1]
  for s in range(S):
    outs[2 + s][...] = r0[:, s * L:(s + 1) * L]


def _layer_body(nq, S_next, *refs):
  h_ref = refs[0]
  sdeg = refs[1][...]
  inv = refs[2][...]
  rrefs = refs[3:3 + nq]
  w_ref = refs[3 + nq]
  b_ref = refs[4 + nq]
  hout_ref = refs[5 + nq] if S_next else refs[7 + nq]
  acc = jnp.dot(h_ref[...], w_ref[0], preferred_element_type=jnp.float32)
  s1 = sdeg[:, :1]
  for j in range(nq):
    acc = acc + jnp.dot(rrefs[j][...] * s1, w_ref[j + 1],
                        preferred_element_type=jnp.float32)
  a = acc + b_ref[0]
  hout = jnp.where(a > 0, a, jnp.exp(jnp.minimum(a, 0.0)) - 1.0)
  if S_next:
    hout_ref[...] = hout
    r_next = hout * (sdeg * inv)[:, :1]
    for s in range(S_next):
      refs[6 + nq + s][...] = r_next[:, s * L:(s + 1) * L]
  else:
    wfc_ref = refs[5 + nq]
    bfc_ref = refs[6 + nq]
    o = jnp.dot(hout, wfc_ref[...], preferred_element_type=jnp.float32) \
        + bfc_ref[0]
    hout_ref[...] = jnp.maximum(o, 0.0)


def _fc_body(h_ref, w_ref, b_ref, out_ref):
  a = jnp.dot(h_ref[...], w_ref[...], preferred_element_type=jnp.float32) \
      + b_ref[0]
  out_ref[...] = jnp.maximum(a, 0.0)


def _row_spec(shape):
  nd = len(shape)
  blk = (BN,) + shape[1:]
  return pl.BlockSpec(blk, lambda i: (i,) + (0,) * (nd - 1))


def _full_spec(shape):
  nd = len(shape)
  return pl.BlockSpec(shape, lambda i: (0,) * nd)


def _tc_call(body, ins, outs):
  grid = (ins[0].shape[0] // BN,)
  in_specs = []
  for a in ins:
    if a.shape[0] % BN == 0 and a.ndim >= 2 and a.shape[0] > BN:
      in_specs.append(_row_spec(a.shape))
    else:
      in_specs.append(_full_spec(a.shape))
  out_specs = [_row_spec(o.shape) for o in outs]
  res = pl.pallas_call(
      body,
      grid=grid,
      in_specs=in_specs,
      out_specs=out_specs,
      out_shape=outs,
  )(*ins)
  return list(res)


def kernel(x, edge_index, W1, b1, W2, b2, Wfc, bfc):
  N, D_IN = x.shape
  E = edge_index.shape[1]
  K1 = W1.shape[0] - 1
  D_H1 = W1.shape[2]
  D_H2 = W2.shape[2]

  per_tile = -(-(N + L) // (NTILES * 800)) * 800
  NP = per_tile * NTILES
  ZB = per_tile // 25
  EP = -(-E // (2 * NTILES * CHUNK_ROWS * ROW_W)) * (2 * NTILES * CHUNK_ROWS
                                                     * ROW_W)
  S1 = -(-D_IN // L)
  S2 = -(-D_H1 // L)
  D1P = S1 * L
  D2P = S2 * L

  src = edge_index[0]
  dst = edge_index[1]
  npad = EP - E
  padi = (jnp.arange(npad, dtype=jnp.int32) % L) + N
  src_1d = jnp.concatenate([src, padi])
  dst_1d = jnp.concatenate([dst, padi])
  dst_r = dst_1d.reshape(EP // ROW_W, ROW_W)

  xp = jnp.pad(x, ((0, NP - N), (0, D1P - D_IN)))
  W1p = jnp.pad(W1, ((0, 0), (0, D1P - D_IN), (0, D2P - D_H1)))
  b1p = jnp.pad(b1, (0, D2P - D_H1)).reshape(1, D2P)
  W2p = jnp.pad(W2, ((0, 0), (0, D2P - D_H1), (0, 0)))
  b2p = b2.reshape(1, D_H2)
  Wfcp = jnp.pad(Wfc, ((0, 0), (0, 48 - D_IN)))
  bfcp = jnp.pad(bfc, (0, 48 - D_IN)).reshape(1, 48)

  dega, degb = _make_deg(NP, EP, ZB)(dst_r)

  sl16 = jax.ShapeDtypeStruct((NP, L), jnp.float32)
  prep = _tc_call(functools.partial(_prep_body, S1), [dega, degb, xp],
                  [sl16, sl16] + [sl16] * S1)
  sdeg, inv = prep[0], prep[1]
  tabs0 = prep[2:]

  hop1 = _make_hop(S1, NP, EP, ZB, True)
  hop1l = _make_hop(S1, NP, EP, ZB, False)
  hop2 = _make_hop(S2, NP, EP, ZB, True)
  hop2l = _make_hop(S2, NP, EP, ZB, False)

  def propagate(hopt, hopl, tabs):
    rws = []
    cur = list(tabs)
    for j in range(K1):
      res = (hopt if j + 1 < K1 else hopl)(src_1d, dst_1d, inv, *cur)
      rws.append(res[0])
      cur = res[1:]
    return rws

  r1w = propagate(hop1, hop1l, tabs0)
  louts = _tc_call(
      functools.partial(_layer_body, K1, S2),
      [xp, sdeg, inv] + r1w + [W1p, b1p],
      [jax.ShapeDtypeStruct((NP, D2P), jnp.float32)] + [sl16] * S2)
  h1, tabs1 = louts[0], louts[1:]

  r2w = propagate(hop2, hop2l, tabs1)
  (outp,) = _tc_call(
      functools.partial(_layer_body, K1, 0),
      [h1, sdeg, inv] + r2w + [W2p, b2p, Wfcp, bfcp],
      [jax.ShapeDtypeStruct((NP, 48), jnp.float32)])
  return outp[:N, :D_IN]
